# baseline (speedup 1.0000x reference)
import functools

import jax
import jax.numpy as jnp
from jax import lax
from jax.experimental import pallas as pl
from jax.experimental.pallas import tpu as pltpu
from jax.experimental.pallas import tpu_sc as plsc

_B = 128
_D = 768
_POOL = 36
_K = 5
_HALF = 10
_S = _K * _HALF
_NROWS = _S * _B
_WPA = 8
_RPW = _NROWS // _WPA
_CH = 80
_NCH = _RPW // _CH


def _score_body(x_ref, k2_ref, k3_ref, g0_ref, g1_ref,
                srck2_ref, srcv2_ref, srck3_ref, srcv3_ref, loss_ref,
                gk0_ref, gv0_ref, gk1_ref, gv1_ref):
    x = x_ref[...]
    qn = x / jnp.maximum(jnp.sqrt(jnp.sum(x * x, axis=1, keepdims=True)), 1e-12)
    iota = lax.broadcasted_iota(jnp.int32, (_POOL, _B), 0)
    tmul = (lax.broadcasted_iota(jnp.int32, (_S, _B), 0) % _HALF) * _POOL
    losses = []
    for k_ref, srck_ref, srcv_ref in ((k2_ref, srck2_ref, srcv2_ref),
                                      (k3_ref, srck3_ref, srcv3_ref)):
        kmat = k_ref[...]
        kn = kmat / jnp.maximum(
            jnp.sqrt(jnp.sum(kmat * kmat, axis=1, keepdims=True)), 1e-12)
        cos = lax.dot_general(kn, qn, (((1,), (1,)), ((), ())),
                              preferred_element_type=jnp.float32)
        rowsum = jnp.sum(cos, axis=1, keepdims=True)
        cm = cos
        acc = jnp.float32(0.0)
        picks = []
        for _ in range(_K):
            v = jnp.max(cm, axis=0, keepdims=True)
            pick = jnp.min(jnp.where(cm == v, iota, _POOL), axis=0,
                           keepdims=True)
            sel = iota == pick
            acc = acc + jnp.sum(
                jnp.where(sel, jnp.broadcast_to(rowsum, (_POOL, _B)), 0.0))
            picks.append(pick)
            cm = jnp.where(sel, jnp.float32(-1e30), cm)
        losses.append(1.0 - acc / jnp.float32(_B * _B * _K))
        rep = jnp.concatenate([p for p in picks for _ in range(_HALF)], axis=0)
        srck_ref[...] = tmul + rep
        srcv_ref[...] = tmul + rep + _HALF * _POOL
    loss_ref[...] = jnp.broadcast_to(
        (losses[0] + losses[1]) / jnp.float32(3.0), (1, 1))
    gk0_ref[...] = jnp.broadcast_to(g0_ref[0:3, :][:, None, :], (3, _B, _D))
    gv0_ref[...] = jnp.broadcast_to(g0_ref[3:6, :][:, None, :], (3, _B, _D))
    gk1_ref[...] = jnp.broadcast_to(g1_ref[0:3, :][:, None, :], (3, _B, _D))
    gv1_ref[...] = jnp.broadcast_to(g1_ref[3:6, :][:, None, :], (3, _B, _D))


_score = pl.pallas_call(
    _score_body,
    out_shape=(
        jax.ShapeDtypeStruct((_S, _B), jnp.int32),
        jax.ShapeDtypeStruct((_S, _B), jnp.int32),
        jax.ShapeDtypeStruct((_S, _B), jnp.int32),
        jax.ShapeDtypeStruct((_S, _B), jnp.int32),
        jax.ShapeDtypeStruct((1, 1), jnp.float32),
        jax.ShapeDtypeStruct((3, _B, _D), jnp.float32),
        jax.ShapeDtypeStruct((3, _B, _D), jnp.float32),
        jax.ShapeDtypeStruct((3, _B, _D), jnp.float32),
        jax.ShapeDtypeStruct((3, _B, _D), jnp.float32),
    ),
)


@functools.cache
def _make_gather():
    mesh = plsc.VectorSubcoreMesh(core_axis_name="c", subcore_axis_name="s")
    return functools.partial(
        pl.kernel,
        mesh=mesh,
        out_type=[jax.ShapeDtypeStruct((_NROWS, _D), jnp.float32)] * 4,
        scratch_types=[
            pltpu.VMEM((_RPW,), jnp.int32),
            pltpu.VMEM((_CH, _D), jnp.float32),
            pltpu.VMEM((_CH, _D), jnp.float32),
            pltpu.SemaphoreType.DMA,
            pltpu.SemaphoreType.DMA,
            pltpu.SemaphoreType.DMA,
            pltpu.SemaphoreType.DMA,
        ],
    )(_gather_body)


def _gather_body(t2_hbm, t3_hbm, i0_hbm, i1_hbm, i2_hbm, i3_hbm,
                 o0_hbm, o1_hbm, o2_hbm, o3_hbm,
                 idx_v, buf0_v, buf1_v, gsem0, gsem1, ssem0, ssem1):
    wid = lax.axis_index("s") * 2 + lax.axis_index("c")
    arm = wid // _WPA
    base = (wid % _WPA) * _RPW
    arms = ((t2_hbm, i0_hbm, o0_hbm), (t2_hbm, i1_hbm, o1_hbm),
            (t3_hbm, i2_hbm, o2_hbm), (t3_hbm, i3_hbm, o3_hbm))
    for a, (t_hbm, i_hbm, o_hbm) in enumerate(arms):
        @pl.when(arm == a)
        def _():
            pltpu.sync_copy(i_hbm.at[pl.ds(base, _RPW)], idx_v)
            bufs = (buf0_v, buf1_v)
            gsems = (gsem0, gsem1)
            ssems = (ssem0, ssem1)
            scps = [None, None]
            for c in range(_NCH):
                p = c % 2
                if c >= 2:
                    scps[p].wait()
                gcp = pltpu.async_copy(
                    t_hbm.at[idx_v.at[pl.ds(c * _CH, _CH)]], bufs[p], gsems[p])
                gcp.wait()
                scps[p] = pltpu.async_copy(
                    bufs[p], o_hbm.at[pl.ds(base + c * _CH, _CH)], ssems[p])
            for p in (_NCH % 2, (_NCH + 1) % 2):
                scps[p].wait()


def kernel(x_querry, g_p_0, g_p_1, e_p_2, e_k_2, e_p_3, e_k_3, e_p_4, e_k_4):
    del e_p_4, e_k_4
    (srck2, srcv2, srck3, srcv3, loss11,
     gk0, gv0, gk1, gv1) = _score(x_querry, e_k_2, e_k_3, g_p_0, g_p_1)
    t2 = e_p_2.transpose(1, 0, 2).reshape(_POOL * 2 * _HALF, _D)
    t3 = e_p_3.transpose(1, 0, 2).reshape(_POOL * 2 * _HALF, _D)
    flat = lambda i: i.reshape(_NROWS)
    ek2, ev2, ek3, ev3 = _make_gather()(
        t2, t3, flat(srck2), flat(srcv2), flat(srck3), flat(srcv3))
    out = lambda a: a.reshape(_S, _B, _D).transpose(1, 0, 2)
    g = lambda a: a.transpose(1, 0, 2)
    return (g(gk0), g(gv0), g(gk1), g(gv1), out(ek2), out(ev2), out(ek3),
            out(ev3), loss11[0, 0])


# speedup vs baseline: 1.0490x; 1.0490x over previous
import functools

import jax
import jax.numpy as jnp
from jax import lax
from jax.experimental import pallas as pl
from jax.experimental.pallas import tpu as pltpu
from jax.experimental.pallas import tpu_sc as plsc

_B = 128
_D = 768
_POOL = 36
_K = 5
_HALF = 10
_S = _K * _HALF
_NROWS = _S * _B
_WPA = 8
_RPW = _NROWS // _WPA
_CH = 80
_NCH = _RPW // _CH


def _score_body(x_ref, k2_ref, k3_ref, g0_ref, g1_ref,
                srck2_ref, srcv2_ref, srck3_ref, srcv3_ref, loss_ref,
                gk0_ref, gv0_ref, gk1_ref, gv1_ref):
    x = x_ref[...]
    qn = x / jnp.maximum(jnp.sqrt(jnp.sum(x * x, axis=1, keepdims=True)), 1e-12)
    iota = lax.broadcasted_iota(jnp.int32, (_POOL, _B), 0)
    tmul = (lax.broadcasted_iota(jnp.int32, (_S, _B), 0) % _HALF) * _POOL
    losses = []
    for k_ref, srck_ref, srcv_ref in ((k2_ref, srck2_ref, srcv2_ref),
                                      (k3_ref, srck3_ref, srcv3_ref)):
        kmat = k_ref[...]
        kn = kmat / jnp.maximum(
            jnp.sqrt(jnp.sum(kmat * kmat, axis=1, keepdims=True)), 1e-12)
        cos = lax.dot_general(kn, qn, (((1,), (1,)), ((), ())),
                              preferred_element_type=jnp.float32)
        rowsum = jnp.sum(cos, axis=1, keepdims=True)
        cm = cos
        acc = jnp.float32(0.0)
        picks = []
        for _ in range(_K):
            v = jnp.max(cm, axis=0, keepdims=True)
            pick = jnp.min(jnp.where(cm == v, iota, _POOL), axis=0,
                           keepdims=True)
            sel = iota == pick
            acc = acc + jnp.sum(
                jnp.where(sel, jnp.broadcast_to(rowsum, (_POOL, _B)), 0.0))
            picks.append(pick)
            cm = jnp.where(sel, jnp.float32(-1e30), cm)
        losses.append(1.0 - acc / jnp.float32(_B * _B * _K))
        rep = jnp.concatenate([p for p in picks for _ in range(_HALF)], axis=0)
        srck_ref[...] = tmul + rep
        srcv_ref[...] = tmul + rep + _HALF * _POOL
    loss_ref[...] = jnp.broadcast_to(
        (losses[0] + losses[1]) / jnp.float32(3.0), (1, 1))
    gk0_ref[...] = jnp.broadcast_to(g0_ref[0:3, :][:, None, :], (3, _B, _D))
    gv0_ref[...] = jnp.broadcast_to(g0_ref[3:6, :][:, None, :], (3, _B, _D))
    gk1_ref[...] = jnp.broadcast_to(g1_ref[0:3, :][:, None, :], (3, _B, _D))
    gv1_ref[...] = jnp.broadcast_to(g1_ref[3:6, :][:, None, :], (3, _B, _D))


_score = pl.pallas_call(
    _score_body,
    out_shape=(
        jax.ShapeDtypeStruct((_S, _B), jnp.int32),
        jax.ShapeDtypeStruct((_S, _B), jnp.int32),
        jax.ShapeDtypeStruct((_S, _B), jnp.int32),
        jax.ShapeDtypeStruct((_S, _B), jnp.int32),
        jax.ShapeDtypeStruct((1, 1), jnp.float32),
        jax.ShapeDtypeStruct((3, _B, _D), jnp.float32),
        jax.ShapeDtypeStruct((3, _B, _D), jnp.float32),
        jax.ShapeDtypeStruct((3, _B, _D), jnp.float32),
        jax.ShapeDtypeStruct((3, _B, _D), jnp.float32),
    ),
)


@functools.cache
def _make_gather():
    mesh = plsc.VectorSubcoreMesh(core_axis_name="c", subcore_axis_name="s")
    return functools.partial(
        pl.kernel,
        mesh=mesh,
        out_type=[jax.ShapeDtypeStruct((_NROWS, _D), jnp.float32)] * 4,
        scratch_types=[
            pltpu.VMEM((_RPW,), jnp.int32),
            pltpu.VMEM((_CH, _D), jnp.float32),
            pltpu.VMEM((_CH, _D), jnp.float32),
            pltpu.SemaphoreType.DMA,
            pltpu.SemaphoreType.DMA,
            pltpu.SemaphoreType.DMA,
            pltpu.SemaphoreType.DMA,
        ],
    )(_gather_body)


def _gather_body(t2_hbm, t3_hbm, i0_hbm, i1_hbm, i2_hbm, i3_hbm,
                 o0_hbm, o1_hbm, o2_hbm, o3_hbm,
                 idx_v, buf0_v, buf1_v, gsem0, gsem1, ssem0, ssem1):
    wid = lax.axis_index("s") * 2 + lax.axis_index("c")
    arm = wid // _WPA
    base = (wid % _WPA) * _RPW
    arms = ((t2_hbm, i0_hbm, o0_hbm), (t2_hbm, i1_hbm, o1_hbm),
            (t3_hbm, i2_hbm, o2_hbm), (t3_hbm, i3_hbm, o3_hbm))
    for a, (t_hbm, i_hbm, o_hbm) in enumerate(arms):
        @pl.when(arm == a)
        def _():
            pltpu.sync_copy(i_hbm.at[pl.ds(base, _RPW)], idx_v)
            bufs = (buf0_v, buf1_v)
            gsems = (gsem0, gsem1)
            ssems = (ssem0, ssem1)
            gcps = [None, None]
            scps = [None, None]
            for c in range(_NCH):
                p = c % 2
                if c >= 2:
                    scps[p].wait()
                gcps[p] = pltpu.async_copy(
                    t_hbm.at[idx_v.at[pl.ds(c * _CH, _CH)]], bufs[p], gsems[p])
                if c >= 1:
                    gcps[1 - p].wait()
                    scps[1 - p] = pltpu.async_copy(
                        bufs[1 - p], o_hbm.at[pl.ds(base + (c - 1) * _CH, _CH)],
                        ssems[1 - p])
            pl_ = (_NCH - 1) % 2
            gcps[pl_].wait()
            scps[pl_] = pltpu.async_copy(
                bufs[pl_], o_hbm.at[pl.ds(base + (_NCH - 1) * _CH, _CH)],
                ssems[pl_])
            scps[1 - pl_].wait()
            scps[pl_].wait()


def kernel(x_querry, g_p_0, g_p_1, e_p_2, e_k_2, e_p_3, e_k_3, e_p_4, e_k_4):
    del e_p_4, e_k_4
    (srck2, srcv2, srck3, srcv3, loss11,
     gk0, gv0, gk1, gv1) = _score(x_querry, e_k_2, e_k_3, g_p_0, g_p_1)
    t2 = e_p_2.transpose(1, 0, 2).reshape(_POOL * 2 * _HALF, _D)
    t3 = e_p_3.transpose(1, 0, 2).reshape(_POOL * 2 * _HALF, _D)
    flat = lambda i: i.reshape(_NROWS)
    ek2, ev2, ek3, ev3 = _make_gather()(
        t2, t3, flat(srck2), flat(srcv2), flat(srck3), flat(srcv3))
    out = lambda a: a.reshape(_S, _B, _D).transpose(1, 0, 2)
    g = lambda a: a.transpose(1, 0, 2)
    return (g(gk0), g(gv0), g(gk1), g(gv1), out(ek2), out(ev2), out(ek3),
            out(ev3), loss11[0, 0])
